# baseline (speedup 1.0000x reference)
import functools

import jax
import jax.numpy as jnp
from jax import lax
from jax.experimental import pallas as pl
from jax.experimental.pallas import tpu as pltpu
from jax.experimental.pallas import tpu_sc as plsc

N_NODES = 10000
N_EDGES = 320000
D = 128
L = 16

NC = 2
NS = 16
NW = NC * NS
CH = 128
K = 80
TS = K // 8
E_PAD = NW * K * CH
ROWS = 10240
RPW = ROWS // NS
CROWS = ROWS // CH


def _sc_aggregate(x, src_p, dst_p, zrows):
  mesh = plsc.VectorSubcoreMesh(core_axis_name="c", subcore_axis_name="s")

  @functools.partial(
      pl.kernel,
      mesh=mesh,
      out_type=[
          jax.ShapeDtypeStruct((NC, ROWS, D), jnp.float32),
          jax.ShapeDtypeStruct((NC, NS, CROWS, CH), jnp.float32),
      ],
      scratch_types=[
          pltpu.VMEM((K, CH), jnp.int32),
          pltpu.VMEM((K, CH), jnp.int32),
          pltpu.VMEM((CH, D), jnp.float32),
          pltpu.VMEM((CROWS, CH), jnp.float32),
          pltpu.VMEM_SHARED((ROWS, D), jnp.float32),
          pltpu.SemaphoreType.DMA,
      ],
      compiler_params=pltpu.CompilerParams(needs_layout_passes=False),
  )
  def k(x_h, src_h, dst_h, zr_h, pout_h, cout_h,
        src_v, dst_v, rows_v, cnt_v, acc, sem):
    cid = lax.axis_index("c")
    sid = lax.axis_index("s")
    wid = cid * NS + sid
    base = sid * RPW

    pltpu.sync_copy(zr_h, rows_v)
    zh = [
        pltpu.async_copy(rows_v, acc.at[pl.ds(base + r * CH, CH)], sem)
        for r in range(RPW // CH)
    ]
    zero16 = jnp.zeros((L,), jnp.float32)

    def zstep(v, carry):
      row = lax.shift_right_logical(v, 3)
      col = lax.mul(lax.bitwise_and(v, 7), L)
      cnt_v[row, pl.ds(col, L)] = zero16
      return carry

    lax.fori_loop(0, CROWS * CH // L, zstep, 0)
    pltpu.sync_copy(src_h.at[wid], src_v)
    pltpu.sync_copy(dst_h.at[wid], dst_v)
    for h in zh:
      h.wait()
    plsc.subcore_barrier()

    one16 = jnp.full((L,), 1.0, jnp.float32)

    def step(j, carry):
      pltpu.async_copy(x_h.at[src_v.at[j]], rows_v, sem).wait()
      pltpu.sync_copy(rows_v, acc.at[dst_v.at[j]], add=True)
      for c in range(CH // L):
        dvec = dst_v[j, pl.ds(c * L, L)]
        i0 = lax.shift_right_logical(dvec, 7)
        i1 = lax.bitwise_and(dvec, 127)
        plsc.addupdate_scatter(cnt_v, [i0, i1], one16)
      return carry

    lax.fori_loop(0, K, step, 0)

    pltpu.sync_copy(cnt_v, cout_h.at[cid, sid])
    plsc.subcore_barrier()

    for r in range(RPW // CH):
      pltpu.sync_copy(acc.at[pl.ds(base + r * CH, CH)],
                      pout_h.at[cid, pl.ds(base + r * CH, CH)])

  return k(x, src_p, dst_p, zrows)


def _tc_finish(partials, counts, W, b2):
  def body(p_ref, c_ref, w_ref, b_ref, o_ref):
    s = p_ref[0] + p_ref[1]
    c = jnp.sum(c_ref[...], axis=(0, 1))
    c3 = jnp.broadcast_to(c[:, :, None], (CROWS, CH, D))
    c2 = jnp.reshape(c3, (ROWS, D))
    h = s / jnp.maximum(c2, 1.0)
    o_ref[...] = lax.dot_general(
        h, w_ref[...], (((1,), (1,)), ((), ())),
        preferred_element_type=jnp.float32) + b_ref[...]

  return pl.pallas_call(
      body,
      out_shape=jax.ShapeDtypeStruct((ROWS, D), jnp.float32),
  )(partials, counts, W, b2)


def kernel(x, edge_index, W, b):
  src = edge_index[0]
  dst = edge_index[1]
  pad = E_PAD - N_EDGES
  src_p = jnp.concatenate([src, jnp.zeros((pad,), jnp.int32)]).reshape(NW, K, CH)
  dst_p = jnp.concatenate(
      [dst, jnp.full((pad,), N_NODES, jnp.int32)]).reshape(NW, K, CH)

  zrows = jnp.zeros((CH, D), jnp.float32)

  partials, counts = _sc_aggregate(x, src_p, dst_p, zrows)
  out = _tc_finish(partials, counts, W, b.reshape(1, D))
  return out[:N_NODES]
